# Initial kernel scaffold; baseline (speedup 1.0000x reference)
import jax
import jax.numpy as jnp
from jax.experimental import pallas as pl


def kernel(pos, W_sa0_1, b_sa0_1, W_sa0_2, b_sa0_2, W_sa1_1, b_sa1_1, W_sa1_2, b_sa1_2, W_fp1, b_fp1, W_fp0, b_fp0, W_l0, b_l0, W_l1, b_l1, batch):
    raise NotImplementedError("write your pallas kernel here")


# speedup vs baseline: 11.8755x; 11.8755x over previous
import functools

import jax
import jax.numpy as jnp
import numpy as np
from jax import lax
from jax.experimental import pallas as pl
from jax.experimental.pallas import tpu as pltpu
from jax.experimental.pallas import tpu_sc as plsc

B = 16
P = 2048
M = P // 2
K = 64
R2 = np.float32(0.04)
NWORK = 32
CPW = M // 2
L = 16


def _fps_body(px_ref, py_ref, pz_ref, idx_ref):
    px = px_ref[...]
    py = py_ref[...]
    pz = pz_ref[...]
    lanes = lax.broadcasted_iota(jnp.int32, (B, P), 1)
    cols = lax.broadcasted_iota(jnp.int32, (B, M), 1)
    lx = px[:, 0:1]
    ly = py[:, 0:1]
    lz = pz[:, 0:1]
    dists0 = jnp.full((B, P), jnp.inf, jnp.float32)
    idxs0 = jnp.zeros((B, M), jnp.int32)

    def body(i, carry):
        dists, idxs, lx, ly, lz = carry
        dx = px - lx
        dy = py - ly
        dz = pz - lz
        d = (dx * dx + dy * dy) + dz * dz
        dists = jnp.minimum(dists, d)
        m = jnp.max(dists, axis=1, keepdims=True)
        nxt = jnp.min(jnp.where(dists == m, lanes, P), axis=1, keepdims=True)
        sel = lanes == nxt
        lx = jnp.sum(jnp.where(sel, px, 0.0), axis=1, keepdims=True)
        ly = jnp.sum(jnp.where(sel, py, 0.0), axis=1, keepdims=True)
        lz = jnp.sum(jnp.where(sel, pz, 0.0), axis=1, keepdims=True)
        idxs = jnp.where(cols == i, nxt, idxs)
        return dists, idxs, lx, ly, lz

    carry = (dists0, idxs0, lx, ly, lz)
    _, idxs, _, _, _ = lax.fori_loop(1, M, body, carry)
    idx_ref[...] = idxs


def _fps(posx, posy, posz):
    return pl.pallas_call(
        _fps_body,
        out_shape=jax.ShapeDtypeStruct((B, M), jnp.int32),
    )(posx, posy, posz)


def _ballq_body(posx_hbm, posy_hbm, posz_hbm, idx_hbm, zero_hbm,
                relx_hbm, rely_hbm, relz_hbm, cx_hbm, cy_hbm, cz_hbm,
                px_v, py_v, pz_v, idx_v, relx_v, rely_v, relz_v,
                cx_v, cy_v, cz_v):
    wid = lax.axis_index("s") * 2 + lax.axis_index("c")
    b = wid // 2
    half = wid % 2
    cbase = half * CPW

    pltpu.sync_copy(posx_hbm.at[b], px_v)
    pltpu.sync_copy(posy_hbm.at[b], py_v)
    pltpu.sync_copy(posz_hbm.at[b], pz_v)
    pltpu.sync_copy(idx_hbm.at[b, pl.ds(cbase, CPW)], idx_v)
    pltpu.sync_copy(zero_hbm, relx_v)
    pltpu.sync_copy(zero_hbm, rely_v)
    pltpu.sync_copy(zero_hbm, relz_v)

    lane = lax.iota(jnp.int32, L)
    lane0 = lane == 0

    def center_loop(c, carry):
        cvec = plsc.load_gather(idx_v, [jnp.full((L,), c, jnp.int32)])
        cx = plsc.load_gather(px_v, [cvec])
        cy = plsc.load_gather(py_v, [cvec])
        cz = plsc.load_gather(pz_v, [cvec])
        crow = jnp.full((L,), c, jnp.int32)

        cK = jnp.full((L,), c * K, jnp.int32)

        def chunk_loop(ch, wp):
            jidx = ch * L + lane
            pxc = plsc.load_gather(px_v, [jidx])
            pyc = plsc.load_gather(py_v, [jidx])
            pzc = plsc.load_gather(pz_v, [jidx])
            dx = pxc - cx
            dy = pyc - cy
            dz = pzc - cz
            d2 = (dx * dx + dy * dy) + dz * dz
            hit = d2 <= R2

            def do_store(wp):
                cum = plsc.cumsum(hit.astype(jnp.int32))
                slots = cK + wp + cum - 1
                plsc.store_scatter(relx_v, [slots], dx, mask=hit)
                plsc.store_scatter(rely_v, [slots], dy, mask=hit)
                plsc.store_scatter(relz_v, [slots], dz, mask=hit)
                cnt = plsc.all_reduce_population_count(hit)
                return wp + cnt

            return lax.cond(jnp.any(hit), do_store, lambda wp: wp, wp)

        lax.fori_loop(0, P // L, chunk_loop, jnp.zeros((L,), jnp.int32))
        plsc.store_scatter(cx_v, [crow], cx, mask=lane0)
        plsc.store_scatter(cy_v, [crow], cy, mask=lane0)
        plsc.store_scatter(cz_v, [crow], cz, mask=lane0)
        return carry

    lax.fori_loop(0, CPW, center_loop, jnp.int32(0))

    pltpu.sync_copy(relx_v, relx_hbm.at[b, pl.ds(cbase * K, CPW * K)])
    pltpu.sync_copy(rely_v, rely_hbm.at[b, pl.ds(cbase * K, CPW * K)])
    pltpu.sync_copy(relz_v, relz_hbm.at[b, pl.ds(cbase * K, CPW * K)])
    pltpu.sync_copy(cx_v, cx_hbm.at[b, pl.ds(cbase, CPW)])
    pltpu.sync_copy(cy_v, cy_hbm.at[b, pl.ds(cbase, CPW)])
    pltpu.sync_copy(cz_v, cz_hbm.at[b, pl.ds(cbase, CPW)])


def _ballq(posx, posy, posz, idx):
    zero = jnp.zeros((CPW * K,), jnp.float32)
    mesh = plsc.VectorSubcoreMesh(core_axis_name="c", subcore_axis_name="s")
    f32 = jnp.float32
    out_type = (
        jax.ShapeDtypeStruct((B, M * K), f32),
        jax.ShapeDtypeStruct((B, M * K), f32),
        jax.ShapeDtypeStruct((B, M * K), f32),
        jax.ShapeDtypeStruct((B, M), f32),
        jax.ShapeDtypeStruct((B, M), f32),
        jax.ShapeDtypeStruct((B, M), f32),
    )
    scratch = [
        pltpu.VMEM((P,), f32),
        pltpu.VMEM((P,), f32),
        pltpu.VMEM((P,), f32),
        pltpu.VMEM((CPW,), jnp.int32),
        pltpu.VMEM((CPW * K,), f32),
        pltpu.VMEM((CPW * K,), f32),
        pltpu.VMEM((CPW * K,), f32),
        pltpu.VMEM((CPW,), f32),
        pltpu.VMEM((CPW,), f32),
        pltpu.VMEM((CPW,), f32),
    ]
    kern = pl.kernel(
        _ballq_body, mesh=mesh, out_type=out_type, scratch_types=scratch,
        compiler_params=pltpu.CompilerParams(needs_layout_passes=False))
    return kern(posx, posy, posz, idx, zero)


def _mlp_body(relx_ref, rely_ref, relz_ref, w1_ref, b1_ref, w2_ref, b2_ref,
              out_ref):
    rx = relx_ref[0][:, :, None]
    ry = rely_ref[0][:, :, None]
    rz = relz_ref[0][:, :, None]
    w1x = w1_ref[0:1, :].reshape(1, 1, 64)
    w1y = w1_ref[1:2, :].reshape(1, 1, 64)
    w1z = w1_ref[2:3, :].reshape(1, 1, 64)
    b1 = b1_ref[...].reshape(1, 1, 64)
    h1 = jnp.maximum(rx * w1x + ry * w1y + rz * w1z + b1, 0.0)
    h1 = h1.reshape(128 * K, 64)
    h2 = jnp.dot(h1, w2_ref[...], preferred_element_type=jnp.float32)
    h2 = jnp.maximum(h2 + b2_ref[...], 0.0)
    out_ref[0] = jnp.max(h2.reshape(128, K, 128), axis=1)


def _mlp_max(relx, rely, relz, W1, b1, W2, b2):
    grid = (B, M // 128)
    return pl.pallas_call(
        _mlp_body,
        grid=grid,
        in_specs=[
            pl.BlockSpec((1, 128, K), lambda b, t: (b, t, 0)),
            pl.BlockSpec((1, 128, K), lambda b, t: (b, t, 0)),
            pl.BlockSpec((1, 128, K), lambda b, t: (b, t, 0)),
            pl.BlockSpec((3, 64), lambda b, t: (0, 0)),
            pl.BlockSpec((1, 64), lambda b, t: (0, 0)),
            pl.BlockSpec((64, 128), lambda b, t: (0, 0)),
            pl.BlockSpec((1, 128), lambda b, t: (0, 0)),
        ],
        out_specs=pl.BlockSpec((1, 128, 128), lambda b, t: (b, t, 0)),
        out_shape=jax.ShapeDtypeStruct((B, M, 128), jnp.float32),
    )(relx, rely, relz, W1, b1.reshape(1, 64), W2, b2.reshape(1, 128))


def _interp_body(pos_ref, cx_ref, cy_ref, cz_ref,
                 x0_ref, wf_ref, bf_ref, wl0_ref, bl0_ref, wl1_ref, bl1_ref,
                 out_ref):
    T = 256
    pt = pos_ref[0]
    tx = pt[:, 0:1]
    ty = pt[:, 1:2]
    tz = pt[:, 2:3]
    cx = cx_ref[0]
    cy = cy_ref[0]
    cz = cz_ref[0]
    tt = (tx * tx + ty * ty) + tz * tz
    cc = (cx * cx + cy * cy) + cz * cz
    ones_t = jnp.ones((T, 1), jnp.float32)
    zeros_t = jnp.zeros((T, 1), jnp.float32)
    A = jnp.concatenate([tx, ty, tz, tt, ones_t, zeros_t, zeros_t, zeros_t],
                        axis=1)
    ones_c = jnp.ones((1, M), jnp.float32)
    zeros_c = jnp.zeros((1, M), jnp.float32)
    Bt = jnp.concatenate([-2.0 * cx, -2.0 * cy, -2.0 * cz, ones_c, cc,
                          zeros_c, zeros_c, zeros_c], axis=0)
    d2 = jnp.dot(A, Bt, preferred_element_type=jnp.float32)

    lanes = lax.broadcasted_iota(jnp.int32, (T, M), 1)
    Wacc = jnp.zeros((T, M), jnp.float32)
    wsum = jnp.zeros((T, 1), jnp.float32)
    dw = d2
    for _ in range(3):
        mval = jnp.min(dw, axis=1, keepdims=True)
        nidx = jnp.min(jnp.where(dw == mval, lanes, M), axis=1, keepdims=True)
        wk = 1.0 / jnp.maximum(mval, 1e-16)
        sel = lanes == nidx
        Wacc = Wacc + jnp.where(sel, wk, 0.0)
        wsum = wsum + wk
        dw = jnp.where(sel, jnp.inf, dw)
    Wacc = Wacc / wsum

    x0u = jnp.dot(Wacc, x0_ref[0], preferred_element_type=jnp.float32)
    h = jnp.maximum(jnp.dot(x0u, wf_ref[...],
                            preferred_element_type=jnp.float32) + bf_ref[...],
                    0.0)
    h = jnp.maximum(jnp.dot(h, wl0_ref[...],
                            preferred_element_type=jnp.float32) + bl0_ref[...],
                    0.0)
    out = jnp.dot(h, wl1_ref[...], preferred_element_type=jnp.float32) \
        + bl1_ref[...]
    out_ref[0] = out


def _interp_head(posb, cx, cy, cz, x0, W_fp0, b_fp0,
                 W_l0, b_l0, W_l1, b_l1):
    grid = (B, P // 256)
    cx3 = cx.reshape(B, 1, M)
    cy3 = cy.reshape(B, 1, M)
    cz3 = cz.reshape(B, 1, M)
    return pl.pallas_call(
        _interp_body,
        grid=grid,
        in_specs=[
            pl.BlockSpec((1, 256, 3), lambda b, t: (b, t, 0)),
            pl.BlockSpec((1, 1, M), lambda b, t: (b, 0, 0)),
            pl.BlockSpec((1, 1, M), lambda b, t: (b, 0, 0)),
            pl.BlockSpec((1, 1, M), lambda b, t: (b, 0, 0)),
            pl.BlockSpec((1, M, 128), lambda b, t: (b, 0, 0)),
            pl.BlockSpec((128, 64), lambda b, t: (0, 0)),
            pl.BlockSpec((1, 64), lambda b, t: (0, 0)),
            pl.BlockSpec((64, 64), lambda b, t: (0, 0)),
            pl.BlockSpec((1, 64), lambda b, t: (0, 0)),
            pl.BlockSpec((64, 10), lambda b, t: (0, 0)),
            pl.BlockSpec((1, 10), lambda b, t: (0, 0)),
        ],
        out_specs=pl.BlockSpec((1, 256, 10), lambda b, t: (b, t, 0)),
        out_shape=jax.ShapeDtypeStruct((B, P, 10), jnp.float32),
    )(posb, cx3, cy3, cz3, x0, W_fp0, b_fp0.reshape(1, 64),
      W_l0, b_l0.reshape(1, 64), W_l1, b_l1.reshape(1, 10))


def kernel(pos, W_sa0_1, b_sa0_1, W_sa0_2, b_sa0_2, W_sa1_1, b_sa1_1,
           W_sa1_2, b_sa1_2, W_fp1, b_fp1, W_fp0, b_fp0, W_l0, b_l0,
           W_l1, b_l1, batch):
    posb = pos.reshape(B, P, 3)
    posx = posb[:, :, 0]
    posy = posb[:, :, 1]
    posz = posb[:, :, 2]

    idx = _fps(posx, posy, posz)
    relx, rely, relz, cx, cy, cz = _ballq(posx, posy, posz, idx)
    relx = relx.reshape(B, M, K)
    rely = rely.reshape(B, M, K)
    relz = relz.reshape(B, M, K)
    x0 = _mlp_max(relx, rely, relz, W_sa0_1, b_sa0_1, W_sa0_2, b_sa0_2)
    out = _interp_head(posb, cx, cy, cz, x0,
                       W_fp0, b_fp0, W_l0, b_l0, W_l1, b_l1)
    return out.reshape(B * P, 10)


# speedup vs baseline: 12.1238x; 1.0209x over previous
import functools

import jax
import jax.numpy as jnp
import numpy as np
from jax import lax
from jax.experimental import pallas as pl
from jax.experimental.pallas import tpu as pltpu
from jax.experimental.pallas import tpu_sc as plsc

B = 16
P = 2048
M = P // 2
K = 64
R2 = np.float32(0.04)
NWORK = 32
CPW = M // 2
L = 16


def _fps_body(px_ref, py_ref, pz_ref, idx_ref):
    px = px_ref[...]
    py = py_ref[...]
    pz = pz_ref[...]
    lanes = lax.broadcasted_iota(jnp.int32, (B, P), 1)
    cols = lax.broadcasted_iota(jnp.int32, (B, M), 1)
    lx = px[:, 0:1]
    ly = py[:, 0:1]
    lz = pz[:, 0:1]
    dists0 = jnp.full((B, P), jnp.inf, jnp.float32)
    idxs0 = jnp.zeros((B, M), jnp.int32)

    def body(i, carry):
        dists, idxs, lx, ly, lz = carry
        dx = px - lx
        dy = py - ly
        dz = pz - lz
        d = (dx * dx + dy * dy) + dz * dz
        dists = jnp.minimum(dists, d)
        m = jnp.max(dists, axis=1, keepdims=True)
        nxt = jnp.min(jnp.where(dists == m, lanes, P), axis=1, keepdims=True)
        sel = lanes == nxt
        lx = jnp.sum(jnp.where(sel, px, 0.0), axis=1, keepdims=True)
        ly = jnp.sum(jnp.where(sel, py, 0.0), axis=1, keepdims=True)
        lz = jnp.sum(jnp.where(sel, pz, 0.0), axis=1, keepdims=True)
        idxs = jnp.where(cols == i, nxt, idxs)
        return dists, idxs, lx, ly, lz

    carry = (dists0, idxs0, lx, ly, lz)
    _, idxs, _, _, _ = lax.fori_loop(1, M, body, carry)
    idx_ref[...] = idxs


def _fps(posx, posy, posz):
    return pl.pallas_call(
        _fps_body,
        out_shape=jax.ShapeDtypeStruct((B, M), jnp.int32),
    )(posx, posy, posz)


def _ballq_body(posx_hbm, posy_hbm, posz_hbm, idx_hbm, zero_hbm,
                relx_hbm, rely_hbm, relz_hbm, cx_hbm, cy_hbm, cz_hbm,
                px_v, py_v, pz_v, idx_v, relx_v, rely_v, relz_v,
                cx_v, cy_v, cz_v):
    wid = lax.axis_index("s") * 2 + lax.axis_index("c")
    b = wid // 2
    half = wid % 2
    cbase = half * CPW

    pltpu.sync_copy(posx_hbm.at[b], px_v)
    pltpu.sync_copy(posy_hbm.at[b], py_v)
    pltpu.sync_copy(posz_hbm.at[b], pz_v)
    pltpu.sync_copy(idx_hbm.at[b, pl.ds(cbase, CPW)], idx_v)
    pltpu.sync_copy(zero_hbm, relx_v)
    pltpu.sync_copy(zero_hbm, rely_v)
    pltpu.sync_copy(zero_hbm, relz_v)

    lane = lax.iota(jnp.int32, L)
    lane0 = lane == 0

    @plsc.parallel_loop(0, CPW, unroll=2)
    def center_loop(c):
        cvec = plsc.load_gather(idx_v, [jnp.full((L,), c, jnp.int32)])
        cx = plsc.load_gather(px_v, [cvec])
        cy = plsc.load_gather(py_v, [cvec])
        cz = plsc.load_gather(pz_v, [cvec])
        crow = jnp.full((L,), c, jnp.int32)

        cK = jnp.full((L,), c * K, jnp.int32)

        def chunk_loop(ch, wp):
            pxc = px_v[pl.ds(ch * L, L)]
            pyc = py_v[pl.ds(ch * L, L)]
            pzc = pz_v[pl.ds(ch * L, L)]
            dx = pxc - cx
            dy = pyc - cy
            dz = pzc - cz
            d2 = (dx * dx + dy * dy) + dz * dz
            hit = d2 <= R2

            def do_store(wp):
                cum = plsc.cumsum(hit.astype(jnp.int32))
                slots = cK + wp + cum - 1
                plsc.store_scatter(relx_v, [slots], dx, mask=hit)
                plsc.store_scatter(rely_v, [slots], dy, mask=hit)
                plsc.store_scatter(relz_v, [slots], dz, mask=hit)
                cnt = plsc.all_reduce_population_count(hit)
                return wp + cnt

            return lax.cond(jnp.any(hit), do_store, lambda wp: wp, wp)

        lax.fori_loop(0, P // L, chunk_loop, jnp.zeros((L,), jnp.int32))
        plsc.store_scatter(cx_v, [crow], cx, mask=lane0)
        plsc.store_scatter(cy_v, [crow], cy, mask=lane0)
        plsc.store_scatter(cz_v, [crow], cz, mask=lane0)

    pltpu.sync_copy(relx_v, relx_hbm.at[b, pl.ds(cbase * K, CPW * K)])
    pltpu.sync_copy(rely_v, rely_hbm.at[b, pl.ds(cbase * K, CPW * K)])
    pltpu.sync_copy(relz_v, relz_hbm.at[b, pl.ds(cbase * K, CPW * K)])
    pltpu.sync_copy(cx_v, cx_hbm.at[b, pl.ds(cbase, CPW)])
    pltpu.sync_copy(cy_v, cy_hbm.at[b, pl.ds(cbase, CPW)])
    pltpu.sync_copy(cz_v, cz_hbm.at[b, pl.ds(cbase, CPW)])


def _ballq(posx, posy, posz, idx):
    zero = jnp.zeros((CPW * K,), jnp.float32)
    mesh = plsc.VectorSubcoreMesh(core_axis_name="c", subcore_axis_name="s")
    f32 = jnp.float32
    out_type = (
        jax.ShapeDtypeStruct((B, M * K), f32),
        jax.ShapeDtypeStruct((B, M * K), f32),
        jax.ShapeDtypeStruct((B, M * K), f32),
        jax.ShapeDtypeStruct((B, M), f32),
        jax.ShapeDtypeStruct((B, M), f32),
        jax.ShapeDtypeStruct((B, M), f32),
    )
    scratch = [
        pltpu.VMEM((P,), f32),
        pltpu.VMEM((P,), f32),
        pltpu.VMEM((P,), f32),
        pltpu.VMEM((CPW,), jnp.int32),
        pltpu.VMEM((CPW * K,), f32),
        pltpu.VMEM((CPW * K,), f32),
        pltpu.VMEM((CPW * K,), f32),
        pltpu.VMEM((CPW,), f32),
        pltpu.VMEM((CPW,), f32),
        pltpu.VMEM((CPW,), f32),
    ]
    kern = pl.kernel(
        _ballq_body, mesh=mesh, out_type=out_type, scratch_types=scratch,
        compiler_params=pltpu.CompilerParams(needs_layout_passes=False))
    return kern(posx, posy, posz, idx, zero)


def _mlp_body(relx_ref, rely_ref, relz_ref, w1_ref, b1_ref, w2_ref, b2_ref,
              out_ref):
    rx = relx_ref[0][:, :, None]
    ry = rely_ref[0][:, :, None]
    rz = relz_ref[0][:, :, None]
    w1x = w1_ref[0:1, :].reshape(1, 1, 64)
    w1y = w1_ref[1:2, :].reshape(1, 1, 64)
    w1z = w1_ref[2:3, :].reshape(1, 1, 64)
    b1 = b1_ref[...].reshape(1, 1, 64)
    h1 = jnp.maximum(rx * w1x + ry * w1y + rz * w1z + b1, 0.0)
    h1 = h1.reshape(128 * K, 64)
    h2 = jnp.dot(h1, w2_ref[...], preferred_element_type=jnp.float32)
    h2 = jnp.maximum(h2 + b2_ref[...], 0.0)
    out_ref[0] = jnp.max(h2.reshape(128, K, 128), axis=1)


def _mlp_max(relx, rely, relz, W1, b1, W2, b2):
    grid = (B, M // 128)
    return pl.pallas_call(
        _mlp_body,
        grid=grid,
        in_specs=[
            pl.BlockSpec((1, 128, K), lambda b, t: (b, t, 0)),
            pl.BlockSpec((1, 128, K), lambda b, t: (b, t, 0)),
            pl.BlockSpec((1, 128, K), lambda b, t: (b, t, 0)),
            pl.BlockSpec((3, 64), lambda b, t: (0, 0)),
            pl.BlockSpec((1, 64), lambda b, t: (0, 0)),
            pl.BlockSpec((64, 128), lambda b, t: (0, 0)),
            pl.BlockSpec((1, 128), lambda b, t: (0, 0)),
        ],
        out_specs=pl.BlockSpec((1, 128, 128), lambda b, t: (b, t, 0)),
        out_shape=jax.ShapeDtypeStruct((B, M, 128), jnp.float32),
    )(relx, rely, relz, W1, b1.reshape(1, 64), W2, b2.reshape(1, 128))


def _interp_body(pos_ref, cx_ref, cy_ref, cz_ref,
                 x0_ref, wf_ref, bf_ref, wl0_ref, bl0_ref, wl1_ref, bl1_ref,
                 out_ref):
    T = 256
    pt = pos_ref[0]
    tx = pt[:, 0:1]
    ty = pt[:, 1:2]
    tz = pt[:, 2:3]
    cx = cx_ref[0]
    cy = cy_ref[0]
    cz = cz_ref[0]
    tt = (tx * tx + ty * ty) + tz * tz
    cc = (cx * cx + cy * cy) + cz * cz
    ones_t = jnp.ones((T, 1), jnp.float32)
    zeros_t = jnp.zeros((T, 1), jnp.float32)
    A = jnp.concatenate([tx, ty, tz, tt, ones_t, zeros_t, zeros_t, zeros_t],
                        axis=1)
    ones_c = jnp.ones((1, M), jnp.float32)
    zeros_c = jnp.zeros((1, M), jnp.float32)
    Bt = jnp.concatenate([-2.0 * cx, -2.0 * cy, -2.0 * cz, ones_c, cc,
                          zeros_c, zeros_c, zeros_c], axis=0)
    d2 = jnp.dot(A, Bt, preferred_element_type=jnp.float32)

    lanes = lax.broadcasted_iota(jnp.int32, (T, M), 1)
    Wacc = jnp.zeros((T, M), jnp.float32)
    wsum = jnp.zeros((T, 1), jnp.float32)
    dw = d2
    for _ in range(3):
        mval = jnp.min(dw, axis=1, keepdims=True)
        nidx = jnp.min(jnp.where(dw == mval, lanes, M), axis=1, keepdims=True)
        wk = 1.0 / jnp.maximum(mval, 1e-16)
        sel = lanes == nidx
        Wacc = Wacc + jnp.where(sel, wk, 0.0)
        wsum = wsum + wk
        dw = jnp.where(sel, jnp.inf, dw)
    Wacc = Wacc / wsum

    x0u = jnp.dot(Wacc, x0_ref[0], preferred_element_type=jnp.float32)
    h = jnp.maximum(jnp.dot(x0u, wf_ref[...],
                            preferred_element_type=jnp.float32) + bf_ref[...],
                    0.0)
    h = jnp.maximum(jnp.dot(h, wl0_ref[...],
                            preferred_element_type=jnp.float32) + bl0_ref[...],
                    0.0)
    out = jnp.dot(h, wl1_ref[...], preferred_element_type=jnp.float32) \
        + bl1_ref[...]
    out_ref[0] = out


def _interp_head(posb, cx, cy, cz, x0, W_fp0, b_fp0,
                 W_l0, b_l0, W_l1, b_l1):
    grid = (B, P // 256)
    cx3 = cx.reshape(B, 1, M)
    cy3 = cy.reshape(B, 1, M)
    cz3 = cz.reshape(B, 1, M)
    return pl.pallas_call(
        _interp_body,
        grid=grid,
        in_specs=[
            pl.BlockSpec((1, 256, 3), lambda b, t: (b, t, 0)),
            pl.BlockSpec((1, 1, M), lambda b, t: (b, 0, 0)),
            pl.BlockSpec((1, 1, M), lambda b, t: (b, 0, 0)),
            pl.BlockSpec((1, 1, M), lambda b, t: (b, 0, 0)),
            pl.BlockSpec((1, M, 128), lambda b, t: (b, 0, 0)),
            pl.BlockSpec((128, 64), lambda b, t: (0, 0)),
            pl.BlockSpec((1, 64), lambda b, t: (0, 0)),
            pl.BlockSpec((64, 64), lambda b, t: (0, 0)),
            pl.BlockSpec((1, 64), lambda b, t: (0, 0)),
            pl.BlockSpec((64, 10), lambda b, t: (0, 0)),
            pl.BlockSpec((1, 10), lambda b, t: (0, 0)),
        ],
        out_specs=pl.BlockSpec((1, 256, 10), lambda b, t: (b, t, 0)),
        out_shape=jax.ShapeDtypeStruct((B, P, 10), jnp.float32),
    )(posb, cx3, cy3, cz3, x0, W_fp0, b_fp0.reshape(1, 64),
      W_l0, b_l0.reshape(1, 64), W_l1, b_l1.reshape(1, 10))


def kernel(pos, W_sa0_1, b_sa0_1, W_sa0_2, b_sa0_2, W_sa1_1, b_sa1_1,
           W_sa1_2, b_sa1_2, W_fp1, b_fp1, W_fp0, b_fp0, W_l0, b_l0,
           W_l1, b_l1, batch):
    posb = pos.reshape(B, P, 3)
    posx = posb[:, :, 0]
    posy = posb[:, :, 1]
    posz = posb[:, :, 2]

    idx = _fps(posx, posy, posz)
    relx, rely, relz, cx, cy, cz = _ballq(posx, posy, posz, idx)
    relx = relx.reshape(B, M, K)
    rely = rely.reshape(B, M, K)
    relz = relz.reshape(B, M, K)
    x0 = _mlp_max(relx, rely, relz, W_sa0_1, b_sa0_1, W_sa0_2, b_sa0_2)
    out = _interp_head(posb, cx, cy, cz, x0,
                       W_fp0, b_fp0, W_l0, b_l0, W_l1, b_l1)
    return out.reshape(B * P, 10)


# speedup vs baseline: 20.5627x; 1.6961x over previous
import functools

import jax
import jax.numpy as jnp
import numpy as np
from jax import lax
from jax.experimental import pallas as pl
from jax.experimental.pallas import tpu as pltpu
from jax.experimental.pallas import tpu_sc as plsc

B = 16
P = 2048
M = P // 2
K = 64
R2 = np.float32(0.04)
NWORK = 32
CPW = M // 2
L = 16


def _fps_body(px_ref, py_ref, pz_ref, idx_ref):
    px = px_ref[...]
    py = py_ref[...]
    pz = pz_ref[...]
    lanes = lax.broadcasted_iota(jnp.int32, (B, P), 1)
    cols = lax.broadcasted_iota(jnp.int32, (B, M), 1)
    lx = px[:, 0:1]
    ly = py[:, 0:1]
    lz = pz[:, 0:1]
    dists0 = jnp.full((B, P), jnp.inf, jnp.float32)
    idxs0 = jnp.zeros((B, M), jnp.int32)

    def body(i, carry):
        dists, idxs, lx, ly, lz = carry
        dx = px - lx
        dy = py - ly
        dz = pz - lz
        d = (dx * dx + dy * dy) + dz * dz
        dists = jnp.minimum(dists, d)
        m = jnp.max(dists, axis=1, keepdims=True)
        nxt = jnp.min(jnp.where(dists == m, lanes, P), axis=1, keepdims=True)
        sel = lanes == nxt
        lx = jnp.sum(jnp.where(sel, px, 0.0), axis=1, keepdims=True)
        ly = jnp.sum(jnp.where(sel, py, 0.0), axis=1, keepdims=True)
        lz = jnp.sum(jnp.where(sel, pz, 0.0), axis=1, keepdims=True)
        idxs = jnp.where(cols == i, nxt, idxs)
        return dists, idxs, lx, ly, lz

    carry = (dists0, idxs0, lx, ly, lz)
    _, idxs, _, _, _ = lax.fori_loop(1, M, body, carry)
    idx_ref[...] = idxs


def _fps(posx, posy, posz):
    return pl.pallas_call(
        _fps_body,
        out_shape=jax.ShapeDtypeStruct((B, M), jnp.int32),
    )(posx, posy, posz)


def _ballq_body(posx_hbm, posy_hbm, posz_hbm, idx_hbm, zero_hbm,
                relx_hbm, rely_hbm, relz_hbm, cx_hbm, cy_hbm, cz_hbm,
                px_v, py_v, pz_v, idx_v, relx_v, rely_v, relz_v,
                cx_v, cy_v, cz_v):
    wid = lax.axis_index("s") * 2 + lax.axis_index("c")
    b = wid // 2
    half = wid % 2
    cbase = half * CPW

    pltpu.sync_copy(posx_hbm.at[b], px_v)
    pltpu.sync_copy(posy_hbm.at[b], py_v)
    pltpu.sync_copy(posz_hbm.at[b], pz_v)
    pltpu.sync_copy(idx_hbm.at[b, pl.ds(cbase, CPW)], idx_v)
    pltpu.sync_copy(zero_hbm, relx_v)
    pltpu.sync_copy(zero_hbm, rely_v)
    pltpu.sync_copy(zero_hbm, relz_v)

    lane = lax.iota(jnp.int32, L)
    lane0 = lane == 0

    @plsc.parallel_loop(0, CPW, unroll=2)
    def center_loop(c):
        cvec = plsc.load_gather(idx_v, [jnp.full((L,), c, jnp.int32)])
        cx = plsc.load_gather(px_v, [cvec])
        cy = plsc.load_gather(py_v, [cvec])
        cz = plsc.load_gather(pz_v, [cvec])
        crow = jnp.full((L,), c, jnp.int32)

        cK = jnp.full((L,), c * K, jnp.int32)

        U = 4

        def chunk_loop(g, wp):
            base = g * (U * L)
            ds = []
            hits = []
            for s in range(U):
                pxc = px_v[pl.ds(base + s * L, L)]
                pyc = py_v[pl.ds(base + s * L, L)]
                pzc = pz_v[pl.ds(base + s * L, L)]
                dx = pxc - cx
                dy = pyc - cy
                dz = pzc - cz
                d2 = (dx * dx + dy * dy) + dz * dz
                ds.append((dx, dy, dz))
                hits.append(d2 <= R2)
            anyhit = hits[0] | hits[1] | hits[2] | hits[3]

            def do_store(wp):
                for s in range(U):
                    hit = hits[s]
                    dx, dy, dz = ds[s]
                    cum = plsc.cumsum(hit.astype(jnp.int32))
                    slots = cK + wp + cum - 1
                    plsc.store_scatter(relx_v, [slots], dx, mask=hit)
                    plsc.store_scatter(rely_v, [slots], dy, mask=hit)
                    plsc.store_scatter(relz_v, [slots], dz, mask=hit)
                    wp = wp + plsc.all_reduce_population_count(hit)
                return wp

            return lax.cond(jnp.any(anyhit), do_store, lambda wp: wp, wp)

        lax.fori_loop(0, P // (U * L), chunk_loop, jnp.zeros((L,), jnp.int32))
        plsc.store_scatter(cx_v, [crow], cx, mask=lane0)
        plsc.store_scatter(cy_v, [crow], cy, mask=lane0)
        plsc.store_scatter(cz_v, [crow], cz, mask=lane0)

    pltpu.sync_copy(relx_v, relx_hbm.at[b, pl.ds(cbase * K, CPW * K)])
    pltpu.sync_copy(rely_v, rely_hbm.at[b, pl.ds(cbase * K, CPW * K)])
    pltpu.sync_copy(relz_v, relz_hbm.at[b, pl.ds(cbase * K, CPW * K)])
    pltpu.sync_copy(cx_v, cx_hbm.at[b, pl.ds(cbase, CPW)])
    pltpu.sync_copy(cy_v, cy_hbm.at[b, pl.ds(cbase, CPW)])
    pltpu.sync_copy(cz_v, cz_hbm.at[b, pl.ds(cbase, CPW)])


def _ballq(posx, posy, posz, idx):
    zero = jnp.zeros((CPW * K,), jnp.float32)
    mesh = plsc.VectorSubcoreMesh(core_axis_name="c", subcore_axis_name="s")
    f32 = jnp.float32
    out_type = (
        jax.ShapeDtypeStruct((B, M * K), f32),
        jax.ShapeDtypeStruct((B, M * K), f32),
        jax.ShapeDtypeStruct((B, M * K), f32),
        jax.ShapeDtypeStruct((B, M), f32),
        jax.ShapeDtypeStruct((B, M), f32),
        jax.ShapeDtypeStruct((B, M), f32),
    )
    scratch = [
        pltpu.VMEM((P,), f32),
        pltpu.VMEM((P,), f32),
        pltpu.VMEM((P,), f32),
        pltpu.VMEM((CPW,), jnp.int32),
        pltpu.VMEM((CPW * K,), f32),
        pltpu.VMEM((CPW * K,), f32),
        pltpu.VMEM((CPW * K,), f32),
        pltpu.VMEM((CPW,), f32),
        pltpu.VMEM((CPW,), f32),
        pltpu.VMEM((CPW,), f32),
    ]
    kern = pl.kernel(
        _ballq_body, mesh=mesh, out_type=out_type, scratch_types=scratch,
        compiler_params=pltpu.CompilerParams(needs_layout_passes=False))
    return kern(posx, posy, posz, idx, zero)


def _mlp_body(relx_ref, rely_ref, relz_ref, w1_ref, b1_ref, w2_ref, b2_ref,
              out_ref):
    rx = relx_ref[0][:, :, None]
    ry = rely_ref[0][:, :, None]
    rz = relz_ref[0][:, :, None]
    w1x = w1_ref[0:1, :].reshape(1, 1, 64)
    w1y = w1_ref[1:2, :].reshape(1, 1, 64)
    w1z = w1_ref[2:3, :].reshape(1, 1, 64)
    b1 = b1_ref[...].reshape(1, 1, 64)
    h1 = jnp.maximum(rx * w1x + ry * w1y + rz * w1z + b1, 0.0)
    h1 = h1.reshape(128 * K, 64)
    h2 = jnp.dot(h1, w2_ref[...], preferred_element_type=jnp.float32)
    h2 = jnp.maximum(h2 + b2_ref[...], 0.0)
    out_ref[0] = jnp.max(h2.reshape(128, K, 128), axis=1)


def _mlp_max(relx, rely, relz, W1, b1, W2, b2):
    grid = (B, M // 128)
    return pl.pallas_call(
        _mlp_body,
        grid=grid,
        in_specs=[
            pl.BlockSpec((1, 128, K), lambda b, t: (b, t, 0)),
            pl.BlockSpec((1, 128, K), lambda b, t: (b, t, 0)),
            pl.BlockSpec((1, 128, K), lambda b, t: (b, t, 0)),
            pl.BlockSpec((3, 64), lambda b, t: (0, 0)),
            pl.BlockSpec((1, 64), lambda b, t: (0, 0)),
            pl.BlockSpec((64, 128), lambda b, t: (0, 0)),
            pl.BlockSpec((1, 128), lambda b, t: (0, 0)),
        ],
        out_specs=pl.BlockSpec((1, 128, 128), lambda b, t: (b, t, 0)),
        out_shape=jax.ShapeDtypeStruct((B, M, 128), jnp.float32),
    )(relx, rely, relz, W1, b1.reshape(1, 64), W2, b2.reshape(1, 128))


def _interp_body(pos_ref, cx_ref, cy_ref, cz_ref,
                 x0_ref, wf_ref, bf_ref, wl0_ref, bl0_ref, wl1_ref, bl1_ref,
                 out_ref):
    T = 256
    pt = pos_ref[0]
    tx = pt[:, 0:1]
    ty = pt[:, 1:2]
    tz = pt[:, 2:3]
    cx = cx_ref[0]
    cy = cy_ref[0]
    cz = cz_ref[0]
    tt = (tx * tx + ty * ty) + tz * tz
    cc = (cx * cx + cy * cy) + cz * cz
    ones_t = jnp.ones((T, 1), jnp.float32)
    zeros_t = jnp.zeros((T, 1), jnp.float32)
    A = jnp.concatenate([tx, ty, tz, tt, ones_t, zeros_t, zeros_t, zeros_t],
                        axis=1)
    ones_c = jnp.ones((1, M), jnp.float32)
    zeros_c = jnp.zeros((1, M), jnp.float32)
    Bt = jnp.concatenate([-2.0 * cx, -2.0 * cy, -2.0 * cz, ones_c, cc,
                          zeros_c, zeros_c, zeros_c], axis=0)
    d2 = jnp.dot(A, Bt, preferred_element_type=jnp.float32)

    lanes = lax.broadcasted_iota(jnp.int32, (T, M), 1)
    Wacc = jnp.zeros((T, M), jnp.float32)
    wsum = jnp.zeros((T, 1), jnp.float32)
    dw = d2
    for _ in range(3):
        mval = jnp.min(dw, axis=1, keepdims=True)
        nidx = jnp.min(jnp.where(dw == mval, lanes, M), axis=1, keepdims=True)
        wk = 1.0 / jnp.maximum(mval, 1e-16)
        sel = lanes == nidx
        Wacc = Wacc + jnp.where(sel, wk, 0.0)
        wsum = wsum + wk
        dw = jnp.where(sel, jnp.inf, dw)
    Wacc = Wacc / wsum

    x0u = jnp.dot(Wacc, x0_ref[0], preferred_element_type=jnp.float32)
    h = jnp.maximum(jnp.dot(x0u, wf_ref[...],
                            preferred_element_type=jnp.float32) + bf_ref[...],
                    0.0)
    h = jnp.maximum(jnp.dot(h, wl0_ref[...],
                            preferred_element_type=jnp.float32) + bl0_ref[...],
                    0.0)
    out = jnp.dot(h, wl1_ref[...], preferred_element_type=jnp.float32) \
        + bl1_ref[...]
    out_ref[0] = out


def _interp_head(posb, cx, cy, cz, x0, W_fp0, b_fp0,
                 W_l0, b_l0, W_l1, b_l1):
    grid = (B, P // 256)
    cx3 = cx.reshape(B, 1, M)
    cy3 = cy.reshape(B, 1, M)
    cz3 = cz.reshape(B, 1, M)
    return pl.pallas_call(
        _interp_body,
        grid=grid,
        in_specs=[
            pl.BlockSpec((1, 256, 3), lambda b, t: (b, t, 0)),
            pl.BlockSpec((1, 1, M), lambda b, t: (b, 0, 0)),
            pl.BlockSpec((1, 1, M), lambda b, t: (b, 0, 0)),
            pl.BlockSpec((1, 1, M), lambda b, t: (b, 0, 0)),
            pl.BlockSpec((1, M, 128), lambda b, t: (b, 0, 0)),
            pl.BlockSpec((128, 64), lambda b, t: (0, 0)),
            pl.BlockSpec((1, 64), lambda b, t: (0, 0)),
            pl.BlockSpec((64, 64), lambda b, t: (0, 0)),
            pl.BlockSpec((1, 64), lambda b, t: (0, 0)),
            pl.BlockSpec((64, 10), lambda b, t: (0, 0)),
            pl.BlockSpec((1, 10), lambda b, t: (0, 0)),
        ],
        out_specs=pl.BlockSpec((1, 256, 10), lambda b, t: (b, t, 0)),
        out_shape=jax.ShapeDtypeStruct((B, P, 10), jnp.float32),
    )(posb, cx3, cy3, cz3, x0, W_fp0, b_fp0.reshape(1, 64),
      W_l0, b_l0.reshape(1, 64), W_l1, b_l1.reshape(1, 10))


def kernel(pos, W_sa0_1, b_sa0_1, W_sa0_2, b_sa0_2, W_sa1_1, b_sa1_1,
           W_sa1_2, b_sa1_2, W_fp1, b_fp1, W_fp0, b_fp0, W_l0, b_l0,
           W_l1, b_l1, batch):
    posb = pos.reshape(B, P, 3)
    posx = posb[:, :, 0]
    posy = posb[:, :, 1]
    posz = posb[:, :, 2]

    idx = _fps(posx, posy, posz)
    relx, rely, relz, cx, cy, cz = _ballq(posx, posy, posz, idx)
    relx = relx.reshape(B, M, K)
    rely = rely.reshape(B, M, K)
    relz = relz.reshape(B, M, K)
    x0 = _mlp_max(relx, rely, relz, W_sa0_1, b_sa0_1, W_sa0_2, b_sa0_2)
    out = _interp_head(posb, cx, cy, cz, x0,
                       W_fp0, b_fp0, W_l0, b_l0, W_l1, b_l1)
    return out.reshape(B * P, 10)


# speedup vs baseline: 24.5903x; 1.1959x over previous
import functools

import jax
import jax.numpy as jnp
import numpy as np
from jax import lax
from jax.experimental import pallas as pl
from jax.experimental.pallas import tpu as pltpu
from jax.experimental.pallas import tpu_sc as plsc

B = 16
P = 2048
M = P // 2
K = 64
R2 = np.float32(0.04)
NWORK = 32
CPW = M // 2
L = 16


def _fps_body(px_ref, py_ref, pz_ref, idx_ref):
    px = px_ref[...]
    py = py_ref[...]
    pz = pz_ref[...]
    lanes = lax.broadcasted_iota(jnp.int32, (B, P), 1)
    cols = lax.broadcasted_iota(jnp.int32, (B, M), 1)
    lx = px[:, 0:1]
    ly = py[:, 0:1]
    lz = pz[:, 0:1]
    dists0 = jnp.full((B, P), jnp.inf, jnp.float32)
    idxs0 = jnp.zeros((B, M), jnp.int32)

    def body(i, carry):
        dists, idxs, lx, ly, lz = carry
        dx = px - lx
        dy = py - ly
        dz = pz - lz
        d = (dx * dx + dy * dy) + dz * dz
        dists = jnp.minimum(dists, d)
        m = jnp.max(dists, axis=1, keepdims=True)
        nxt = jnp.min(jnp.where(dists == m, lanes, P), axis=1, keepdims=True)
        sel = lanes == nxt
        lx = jnp.sum(jnp.where(sel, px, 0.0), axis=1, keepdims=True)
        ly = jnp.sum(jnp.where(sel, py, 0.0), axis=1, keepdims=True)
        lz = jnp.sum(jnp.where(sel, pz, 0.0), axis=1, keepdims=True)
        idxs = jnp.where(cols == i, nxt, idxs)
        return dists, idxs, lx, ly, lz

    carry = (dists0, idxs0, lx, ly, lz)
    _, idxs, _, _, _ = lax.fori_loop(1, M, body, carry)
    idx_ref[...] = idxs


def _fps(posx, posy, posz):
    return pl.pallas_call(
        _fps_body,
        out_shape=jax.ShapeDtypeStruct((B, M), jnp.int32),
    )(posx, posy, posz)


def _ballq_body(posx_hbm, posy_hbm, posz_hbm, idx_hbm, zero_hbm,
                relx_hbm, rely_hbm, relz_hbm, cx_hbm, cy_hbm, cz_hbm,
                px_v, py_v, pz_v, idx_v, relx_v, rely_v, relz_v,
                cx_v, cy_v, cz_v):
    wid = lax.axis_index("s") * 2 + lax.axis_index("c")
    b = wid // 2
    half = wid % 2
    cbase = half * CPW

    pltpu.sync_copy(posx_hbm.at[b], px_v)
    pltpu.sync_copy(posy_hbm.at[b], py_v)
    pltpu.sync_copy(posz_hbm.at[b], pz_v)
    pltpu.sync_copy(idx_hbm.at[b, pl.ds(cbase, CPW)], idx_v)
    pltpu.sync_copy(zero_hbm, relx_v)
    pltpu.sync_copy(zero_hbm, rely_v)
    pltpu.sync_copy(zero_hbm, relz_v)

    lane = lax.iota(jnp.int32, L)
    lane0 = lane == 0

    @plsc.parallel_loop(0, CPW, unroll=2)
    def center_loop(c):
        cvec = plsc.load_gather(idx_v, [jnp.full((L,), c, jnp.int32)])
        cx = plsc.load_gather(px_v, [cvec])
        cy = plsc.load_gather(py_v, [cvec])
        cz = plsc.load_gather(pz_v, [cvec])
        crow = jnp.full((L,), c, jnp.int32)

        cK = jnp.full((L,), c * K, jnp.int32)

        U = 8

        def chunk_loop(g, wp):
            base = g * (U * L)
            ds = []
            hits = []
            for s in range(U):
                pxc = px_v[pl.ds(base + s * L, L)]
                pyc = py_v[pl.ds(base + s * L, L)]
                pzc = pz_v[pl.ds(base + s * L, L)]
                dx = pxc - cx
                dy = pyc - cy
                dz = pzc - cz
                d2 = (dx * dx + dy * dy) + dz * dz
                ds.append((dx, dy, dz))
                hits.append(d2 <= R2)
            anyhit = hits[0]
            for s in range(1, U):
                anyhit = anyhit | hits[s]

            def do_store(wp):
                for s in range(U):
                    hit = hits[s]
                    dx, dy, dz = ds[s]
                    cum = plsc.cumsum(hit.astype(jnp.int32))
                    slots = cK + wp + cum - 1
                    plsc.store_scatter(relx_v, [slots], dx, mask=hit)
                    plsc.store_scatter(rely_v, [slots], dy, mask=hit)
                    plsc.store_scatter(relz_v, [slots], dz, mask=hit)
                    wp = wp + plsc.all_reduce_population_count(hit)
                return wp

            return lax.cond(jnp.any(anyhit), do_store, lambda wp: wp, wp)

        lax.fori_loop(0, P // (U * L), chunk_loop, jnp.zeros((L,), jnp.int32))
        plsc.store_scatter(cx_v, [crow], cx, mask=lane0)
        plsc.store_scatter(cy_v, [crow], cy, mask=lane0)
        plsc.store_scatter(cz_v, [crow], cz, mask=lane0)

    pltpu.sync_copy(relx_v, relx_hbm.at[b, pl.ds(cbase * K, CPW * K)])
    pltpu.sync_copy(rely_v, rely_hbm.at[b, pl.ds(cbase * K, CPW * K)])
    pltpu.sync_copy(relz_v, relz_hbm.at[b, pl.ds(cbase * K, CPW * K)])
    pltpu.sync_copy(cx_v, cx_hbm.at[b, pl.ds(cbase, CPW)])
    pltpu.sync_copy(cy_v, cy_hbm.at[b, pl.ds(cbase, CPW)])
    pltpu.sync_copy(cz_v, cz_hbm.at[b, pl.ds(cbase, CPW)])


def _ballq(posx, posy, posz, idx):
    zero = jnp.zeros((CPW * K,), jnp.float32)
    mesh = plsc.VectorSubcoreMesh(core_axis_name="c", subcore_axis_name="s")
    f32 = jnp.float32
    out_type = (
        jax.ShapeDtypeStruct((B, M * K), f32),
        jax.ShapeDtypeStruct((B, M * K), f32),
        jax.ShapeDtypeStruct((B, M * K), f32),
        jax.ShapeDtypeStruct((B, M), f32),
        jax.ShapeDtypeStruct((B, M), f32),
        jax.ShapeDtypeStruct((B, M), f32),
    )
    scratch = [
        pltpu.VMEM((P,), f32),
        pltpu.VMEM((P,), f32),
        pltpu.VMEM((P,), f32),
        pltpu.VMEM((CPW,), jnp.int32),
        pltpu.VMEM((CPW * K,), f32),
        pltpu.VMEM((CPW * K,), f32),
        pltpu.VMEM((CPW * K,), f32),
        pltpu.VMEM((CPW,), f32),
        pltpu.VMEM((CPW,), f32),
        pltpu.VMEM((CPW,), f32),
    ]
    kern = pl.kernel(
        _ballq_body, mesh=mesh, out_type=out_type, scratch_types=scratch,
        compiler_params=pltpu.CompilerParams(needs_layout_passes=False))
    return kern(posx, posy, posz, idx, zero)


def _mlp_body(relx_ref, rely_ref, relz_ref, w1_ref, b1_ref, w2_ref, b2_ref,
              out_ref):
    rx = relx_ref[0][:, :, None]
    ry = rely_ref[0][:, :, None]
    rz = relz_ref[0][:, :, None]
    w1x = w1_ref[0:1, :].reshape(1, 1, 64)
    w1y = w1_ref[1:2, :].reshape(1, 1, 64)
    w1z = w1_ref[2:3, :].reshape(1, 1, 64)
    b1 = b1_ref[...].reshape(1, 1, 64)
    h1 = jnp.maximum(rx * w1x + ry * w1y + rz * w1z + b1, 0.0)
    h1 = h1.reshape(128 * K, 64)
    h2 = jnp.dot(h1, w2_ref[...], preferred_element_type=jnp.float32)
    h2 = jnp.maximum(h2 + b2_ref[...], 0.0)
    out_ref[0] = jnp.max(h2.reshape(128, K, 128), axis=1)


def _mlp_max(relx, rely, relz, W1, b1, W2, b2):
    grid = (B, M // 128)
    return pl.pallas_call(
        _mlp_body,
        grid=grid,
        in_specs=[
            pl.BlockSpec((1, 128, K), lambda b, t: (b, t, 0)),
            pl.BlockSpec((1, 128, K), lambda b, t: (b, t, 0)),
            pl.BlockSpec((1, 128, K), lambda b, t: (b, t, 0)),
            pl.BlockSpec((3, 64), lambda b, t: (0, 0)),
            pl.BlockSpec((1, 64), lambda b, t: (0, 0)),
            pl.BlockSpec((64, 128), lambda b, t: (0, 0)),
            pl.BlockSpec((1, 128), lambda b, t: (0, 0)),
        ],
        out_specs=pl.BlockSpec((1, 128, 128), lambda b, t: (b, t, 0)),
        out_shape=jax.ShapeDtypeStruct((B, M, 128), jnp.float32),
    )(relx, rely, relz, W1, b1.reshape(1, 64), W2, b2.reshape(1, 128))


def _interp_body(pos_ref, cx_ref, cy_ref, cz_ref,
                 x0_ref, wf_ref, bf_ref, wl0_ref, bl0_ref, wl1_ref, bl1_ref,
                 out_ref):
    T = 256
    pt = pos_ref[0]
    tx = pt[:, 0:1]
    ty = pt[:, 1:2]
    tz = pt[:, 2:3]
    cx = cx_ref[0]
    cy = cy_ref[0]
    cz = cz_ref[0]
    tt = (tx * tx + ty * ty) + tz * tz
    cc = (cx * cx + cy * cy) + cz * cz
    ones_t = jnp.ones((T, 1), jnp.float32)
    zeros_t = jnp.zeros((T, 1), jnp.float32)
    A = jnp.concatenate([tx, ty, tz, tt, ones_t, zeros_t, zeros_t, zeros_t],
                        axis=1)
    ones_c = jnp.ones((1, M), jnp.float32)
    zeros_c = jnp.zeros((1, M), jnp.float32)
    Bt = jnp.concatenate([-2.0 * cx, -2.0 * cy, -2.0 * cz, ones_c, cc,
                          zeros_c, zeros_c, zeros_c], axis=0)
    d2 = jnp.dot(A, Bt, preferred_element_type=jnp.float32)

    lanes = lax.broadcasted_iota(jnp.int32, (T, M), 1)
    Wacc = jnp.zeros((T, M), jnp.float32)
    wsum = jnp.zeros((T, 1), jnp.float32)
    dw = d2
    for _ in range(3):
        mval = jnp.min(dw, axis=1, keepdims=True)
        nidx = jnp.min(jnp.where(dw == mval, lanes, M), axis=1, keepdims=True)
        wk = 1.0 / jnp.maximum(mval, 1e-16)
        sel = lanes == nidx
        Wacc = Wacc + jnp.where(sel, wk, 0.0)
        wsum = wsum + wk
        dw = jnp.where(sel, jnp.inf, dw)
    Wacc = Wacc / wsum

    x0u = jnp.dot(Wacc, x0_ref[0], preferred_element_type=jnp.float32)
    h = jnp.maximum(jnp.dot(x0u, wf_ref[...],
                            preferred_element_type=jnp.float32) + bf_ref[...],
                    0.0)
    h = jnp.maximum(jnp.dot(h, wl0_ref[...],
                            preferred_element_type=jnp.float32) + bl0_ref[...],
                    0.0)
    out = jnp.dot(h, wl1_ref[...], preferred_element_type=jnp.float32) \
        + bl1_ref[...]
    out_ref[0] = out


def _interp_head(posb, cx, cy, cz, x0, W_fp0, b_fp0,
                 W_l0, b_l0, W_l1, b_l1):
    grid = (B, P // 256)
    cx3 = cx.reshape(B, 1, M)
    cy3 = cy.reshape(B, 1, M)
    cz3 = cz.reshape(B, 1, M)
    return pl.pallas_call(
        _interp_body,
        grid=grid,
        in_specs=[
            pl.BlockSpec((1, 256, 3), lambda b, t: (b, t, 0)),
            pl.BlockSpec((1, 1, M), lambda b, t: (b, 0, 0)),
            pl.BlockSpec((1, 1, M), lambda b, t: (b, 0, 0)),
            pl.BlockSpec((1, 1, M), lambda b, t: (b, 0, 0)),
            pl.BlockSpec((1, M, 128), lambda b, t: (b, 0, 0)),
            pl.BlockSpec((128, 64), lambda b, t: (0, 0)),
            pl.BlockSpec((1, 64), lambda b, t: (0, 0)),
            pl.BlockSpec((64, 64), lambda b, t: (0, 0)),
            pl.BlockSpec((1, 64), lambda b, t: (0, 0)),
            pl.BlockSpec((64, 10), lambda b, t: (0, 0)),
            pl.BlockSpec((1, 10), lambda b, t: (0, 0)),
        ],
        out_specs=pl.BlockSpec((1, 256, 10), lambda b, t: (b, t, 0)),
        out_shape=jax.ShapeDtypeStruct((B, P, 10), jnp.float32),
    )(posb, cx3, cy3, cz3, x0, W_fp0, b_fp0.reshape(1, 64),
      W_l0, b_l0.reshape(1, 64), W_l1, b_l1.reshape(1, 10))


def kernel(pos, W_sa0_1, b_sa0_1, W_sa0_2, b_sa0_2, W_sa1_1, b_sa1_1,
           W_sa1_2, b_sa1_2, W_fp1, b_fp1, W_fp0, b_fp0, W_l0, b_l0,
           W_l1, b_l1, batch):
    posb = pos.reshape(B, P, 3)
    posx = posb[:, :, 0]
    posy = posb[:, :, 1]
    posz = posb[:, :, 2]

    idx = _fps(posx, posy, posz)
    relx, rely, relz, cx, cy, cz = _ballq(posx, posy, posz, idx)
    relx = relx.reshape(B, M, K)
    rely = rely.reshape(B, M, K)
    relz = relz.reshape(B, M, K)
    x0 = _mlp_max(relx, rely, relz, W_sa0_1, b_sa0_1, W_sa0_2, b_sa0_2)
    out = _interp_head(posb, cx, cy, cz, x0,
                       W_fp0, b_fp0, W_l0, b_l0, W_l1, b_l1)
    return out.reshape(B * P, 10)
